# Initial kernel scaffold; baseline (speedup 1.0000x reference)
import functools

import jax
import jax.numpy as jnp
from jax import lax
from jax.experimental import pallas as pl
from jax.experimental.pallas import tpu as pltpu
from jax.experimental.pallas import tpu_sc as plsc

D_MODEL = 768
MAX_S = 4096

_info = plsc.get_sparse_core_info()
_NC, _NS = _info.num_cores, _info.num_subcores
_NW = _NC * _NS


def _pe_body(o_ref):
    rows = o_ref.shape[0]
    base = pl.program_id(0) * rows
    pos = jax.lax.broadcasted_iota(jnp.float32, (rows, D_MODEL), 0) + base
    j = jax.lax.broadcasted_iota(jnp.int32, (rows, D_MODEL), 1)
    k = (j // 2).astype(jnp.float32)
    ang = pos * jnp.exp(k * (-2.0 * jnp.log(10000.0) / D_MODEL))
    o_ref[...] = jnp.where(j % 2 == 0, jnp.sin(ang), jnp.cos(ang))


def _make_pe(seq_len):
    rows = 512
    return pl.pallas_call(
        _pe_body,
        grid=(seq_len // rows,),
        out_specs=pl.BlockSpec((rows, D_MODEL), lambda i: (i, 0)),
        out_shape=jax.ShapeDtypeStruct((seq_len, D_MODEL), jnp.float32),
    )()


def _make_emb(T, V, D, seq_len):
    assert T % _NW == 0
    tok_w = T // _NW
    C = 32
    assert tok_w % C == 0
    nchunk = tok_w // C
    assert seq_len % tok_w == 0

    mesh = plsc.VectorSubcoreMesh(core_axis_name="c", subcore_axis_name="s")

    @functools.partial(
        pl.kernel,
        mesh=mesh,
        out_type=jax.ShapeDtypeStruct((T, D), jnp.float32),
        scratch_types=[
            pltpu.VMEM((tok_w,), jnp.int32),
            pltpu.VMEM((C, D), jnp.float32),
            pltpu.VMEM((C, D), jnp.float32),
            pltpu.VMEM((C, D), jnp.float32),
            pltpu.VMEM((C, D), jnp.float32),
            pltpu.SemaphoreType.DMA,
            pltpu.SemaphoreType.DMA,
            pltpu.SemaphoreType.DMA,
            pltpu.SemaphoreType.DMA,
        ],
    )
    def emb(x_hbm, table_hbm, pe_hbm, out_hbm,
            idx_v, g0, g1, p0, p1, sg0, sg1, ss0, ss1):
        wid = lax.axis_index("s") * _NC + lax.axis_index("c")
        base = wid * tok_w
        s_base = base % seq_len
        pltpu.sync_copy(x_hbm.at[pl.ds(base, tok_w)], idx_v)

        gbufs = (g0, g1)
        pbufs = (p0, p1)
        gsems = (sg0, sg1)
        ssems = (ss0, ss1)

        def issue(c, b):
            gd = pltpu.async_copy(
                table_hbm.at[idx_v.at[pl.ds(c * C, C)]], gbufs[b], gsems[b])
            pd = pltpu.async_copy(
                pe_hbm.at[pl.ds(s_base + c * C, C)], pbufs[b], gsems[b])
            return gd, pd

        store_desc = [None, None]
        descs = [None, None]
        descs[0] = issue(0, 0)
        for c in range(nchunk):
            b = c & 1
            if c + 1 < nchunk:
                if store_desc[1 - b] is not None:
                    store_desc[1 - b].wait()
                    store_desc[1 - b] = None
                descs[1 - b] = issue(c + 1, 1 - b)
            gd, pd = descs[b]
            gd.wait()
            pd.wait()
            g, p = gbufs[b], pbufs[b]

            @plsc.parallel_loop(0, C)
            def _row(r):
                @plsc.parallel_loop(0, D, step=16, unroll=8)
                def _col(j):
                    g[r, pl.ds(j, 16)] = g[r, pl.ds(j, 16)] + p[r, pl.ds(j, 16)]

            store_desc[b] = pltpu.async_copy(
                g, out_hbm.at[pl.ds(base + c * C, C)], ssems[b])
        for b in range(2):
            if store_desc[b] is not None:
                store_desc[b].wait()

    return emb


def kernel(x, table):
    B, S = x.shape
    V, D = table.shape
    pe = _make_pe(S)
    xf = x.reshape(-1).astype(jnp.int32)
    out = _make_emb(B * S, V, D, S)(xf, table, pe)
    return out.reshape(B, S, D)


# speedup vs baseline: 1.3645x; 1.3645x over previous
import functools

import jax
import jax.numpy as jnp
from jax import lax
from jax.experimental import pallas as pl
from jax.experimental.pallas import tpu as pltpu
from jax.experimental.pallas import tpu_sc as plsc

D_MODEL = 768
MAX_S = 4096

_info = plsc.get_sparse_core_info()
_NC, _NS = _info.num_cores, _info.num_subcores
_NW = _NC * _NS


def _pe_body(o_ref):
    rows = o_ref.shape[0]
    base = pl.program_id(0) * rows
    pos = (jax.lax.broadcasted_iota(jnp.int32, (rows, D_MODEL), 0)
           + base).astype(jnp.float32)
    j = jax.lax.broadcasted_iota(jnp.int32, (rows, D_MODEL), 1)
    k = (j // 2).astype(jnp.float32)
    ang = pos * jnp.exp(k * (-2.0 * jnp.log(10000.0) / D_MODEL))
    o_ref[...] = jnp.where(j % 2 == 0, jnp.sin(ang), jnp.cos(ang))


def _make_pe(seq_len):
    rows = 512
    return pl.pallas_call(
        _pe_body,
        grid=(seq_len // rows,),
        out_specs=pl.BlockSpec((rows, D_MODEL), lambda i: (i, 0)),
        out_shape=jax.ShapeDtypeStruct((seq_len, D_MODEL), jnp.float32),
    )()


def _make_emb(T, V, D, seq_len):
    assert T % _NW == 0
    tok_w = T // _NW
    C = 32
    assert tok_w % C == 0
    nchunk = tok_w // C
    assert seq_len % tok_w == 0

    mesh = plsc.VectorSubcoreMesh(core_axis_name="c", subcore_axis_name="s")

    @functools.partial(
        pl.kernel,
        mesh=mesh,
        out_type=jax.ShapeDtypeStruct((T, D), jnp.float32),
        scratch_types=[
            pltpu.VMEM((tok_w,), jnp.int32),
            pltpu.VMEM((C, D), jnp.float32),
            pltpu.VMEM((C, D), jnp.float32),
            pltpu.VMEM((C, D), jnp.float32),
            pltpu.VMEM((C, D), jnp.float32),
            pltpu.SemaphoreType.DMA,
            pltpu.SemaphoreType.DMA,
            pltpu.SemaphoreType.DMA,
            pltpu.SemaphoreType.DMA,
        ],
    )
    def emb(x_hbm, table_hbm, pe_hbm, out_hbm,
            idx_v, g0, g1, p0, p1, sg0, sg1, ss0, ss1):
        wid = lax.axis_index("s") * _NC + lax.axis_index("c")
        base = wid * tok_w
        s_base = base % seq_len
        pltpu.sync_copy(x_hbm.at[pl.ds(base, tok_w)], idx_v)

        gbufs = (g0, g1)
        pbufs = (p0, p1)
        gsems = (sg0, sg1)
        ssems = (ss0, ss1)

        def issue(c, b):
            gd = pltpu.async_copy(
                table_hbm.at[idx_v.at[pl.ds(c * C, C)]], gbufs[b], gsems[b])
            pd = pltpu.async_copy(
                pe_hbm.at[pl.ds(s_base + c * C, C)], pbufs[b], gsems[b])
            return gd, pd

        store_desc = [None, None]
        descs = [None, None]
        descs[0] = issue(0, 0)
        for c in range(nchunk):
            b = c & 1
            if c + 1 < nchunk:
                if store_desc[1 - b] is not None:
                    store_desc[1 - b].wait()
                    store_desc[1 - b] = None
                descs[1 - b] = issue(c + 1, 1 - b)
            gd, pd = descs[b]
            gd.wait()
            pd.wait()
            g, p = gbufs[b], pbufs[b]

            @plsc.parallel_loop(0, C)
            def _row(r):
                @plsc.parallel_loop(0, D, step=16, unroll=8)
                def _col(j):
                    g[r, pl.ds(j, 16)] = g[r, pl.ds(j, 16)] + p[r, pl.ds(j, 16)]

            store_desc[b] = pltpu.async_copy(
                g, out_hbm.at[pl.ds(base + c * C, C)], ssems[b])
        for b in range(2):
            if store_desc[b] is not None:
                store_desc[b].wait()

    return emb


def kernel(x, table):
    B, S = x.shape
    V, D = table.shape
    pe = _make_pe(S)
    xf = x.reshape(-1).astype(jnp.int32)
    out = _make_emb(B * S, V, D, S)(xf, table, pe)
    return out.reshape(B, S, D)


# speedup vs baseline: 1.4205x; 1.0411x over previous
import functools

import jax
import jax.numpy as jnp
from jax import lax
from jax.experimental import pallas as pl
from jax.experimental.pallas import tpu as pltpu
from jax.experimental.pallas import tpu_sc as plsc

D_MODEL = 768
MAX_S = 4096

_info = plsc.get_sparse_core_info()
_NC, _NS = _info.num_cores, _info.num_subcores
_NW = _NC * _NS


def _pe_body(o_ref):
    rows = o_ref.shape[0]
    base = pl.program_id(0) * rows
    pos = (jax.lax.broadcasted_iota(jnp.int32, (rows, D_MODEL), 0)
           + base).astype(jnp.float32)
    j = jax.lax.broadcasted_iota(jnp.int32, (rows, D_MODEL), 1)
    k = (j // 2).astype(jnp.float32)
    ang = pos * jnp.exp(k * (-2.0 * jnp.log(10000.0) / D_MODEL))
    o_ref[...] = jnp.where(j % 2 == 0, jnp.sin(ang), jnp.cos(ang))


def _make_pe(seq_len):
    rows = 512
    return pl.pallas_call(
        _pe_body,
        grid=(seq_len // rows,),
        out_specs=pl.BlockSpec((rows, D_MODEL), lambda i: (i, 0)),
        out_shape=jax.ShapeDtypeStruct((seq_len, D_MODEL), jnp.float32),
    )()


def _make_emb(B, V, D, seq_len):
    assert seq_len % _NW == 0
    s_w = seq_len // _NW
    Cs = 16
    assert s_w % Cs == 0
    nchunk = s_w // Cs
    R = B * Cs

    mesh = plsc.VectorSubcoreMesh(core_axis_name="c", subcore_axis_name="s")

    @functools.partial(
        pl.kernel,
        mesh=mesh,
        out_type=jax.ShapeDtypeStruct((B * seq_len, D), jnp.float32),
        scratch_types=[
            pltpu.VMEM((B, s_w), jnp.int32),
            pltpu.VMEM((R, D), jnp.float32),
            pltpu.VMEM((R, D), jnp.float32),
            pltpu.VMEM((Cs, D), jnp.float32),
            pltpu.VMEM((Cs, D), jnp.float32),
            pltpu.SemaphoreType.DMA,
            pltpu.SemaphoreType.DMA,
            pltpu.SemaphoreType.DMA,
            pltpu.SemaphoreType.DMA,
        ],
    )
    def emb(x_hbm, table_hbm, pe_hbm, out_hbm,
            idx_v, g0, g1, p0, p1, sg0, sg1, ss0, ss1):
        wid = lax.axis_index("s") * _NC + lax.axis_index("c")
        s_base = wid * s_w
        for b in range(B):
            pltpu.sync_copy(x_hbm.at[pl.ds(b * seq_len + s_base, s_w)],
                            idx_v.at[b])

        gbufs = (g0, g1)
        pbufs = (p0, p1)
        gsems = (sg0, sg1)
        ssems = (ss0, ss1)

        def issue(c, k):
            ds = []
            for b in range(B):
                ds.append(pltpu.async_copy(
                    table_hbm.at[idx_v.at[b, pl.ds(c * Cs, Cs)]],
                    gbufs[k].at[pl.ds(b * Cs, Cs)], gsems[k]))
            ds.append(pltpu.async_copy(
                pe_hbm.at[pl.ds(s_base + c * Cs, Cs)], pbufs[k], gsems[k]))
            return ds

        store_desc = [None, None]
        descs = [None, None]
        descs[0] = issue(0, 0)
        for c in range(nchunk):
            k = c & 1
            if c + 1 < nchunk:
                if store_desc[1 - k] is not None:
                    for sd in store_desc[1 - k]:
                        sd.wait()
                    store_desc[1 - k] = None
                descs[1 - k] = issue(c + 1, 1 - k)
            for d in descs[k]:
                d.wait()
            g, p = gbufs[k], pbufs[k]

            for b in range(B):
                @plsc.parallel_loop(0, Cs)
                def _row(r):
                    @plsc.parallel_loop(0, D, step=16, unroll=8)
                    def _col(j, b=b):
                        g[b * Cs + r, pl.ds(j, 16)] = (
                            g[b * Cs + r, pl.ds(j, 16)] + p[r, pl.ds(j, 16)])

            sds = []
            for b in range(B):
                sds.append(pltpu.async_copy(
                    g.at[pl.ds(b * Cs, Cs)],
                    out_hbm.at[pl.ds(b * seq_len + s_base + c * Cs, Cs)],
                    ssems[k]))
            store_desc[k] = sds
        for k in range(2):
            if store_desc[k] is not None:
                for sd in store_desc[k]:
                    sd.wait()

    return emb


def kernel(x, table):
    B, S = x.shape
    V, D = table.shape
    pe = _make_pe(S)
    xf = x.reshape(-1).astype(jnp.int32)
    out = _make_emb(B, V, D, S)(xf, table, pe)
    return out.reshape(B, S, D)


# speedup vs baseline: 2.0326x; 1.4309x over previous
import functools

import jax
import jax.numpy as jnp
from jax import lax
from jax.experimental import pallas as pl
from jax.experimental.pallas import tpu as pltpu
from jax.experimental.pallas import tpu_sc as plsc

D_MODEL = 768
MAX_S = 4096

try:
    _info = plsc.get_sparse_core_info()
    _NC, _NS = _info.num_cores, _info.num_subcores
except ValueError:
    _NC, _NS = 2, 16
_NW = _NC * _NS


def _pe_body(o_ref):
    rows = o_ref.shape[0]
    hi_n = rows // 64
    base = pl.program_id(0) * rows

    j_hi = jax.lax.broadcasted_iota(jnp.int32, (hi_n, D_MODEL), 1)
    k_hi = (j_hi // 2).astype(jnp.float32)
    div_hi = jnp.exp(k_hi * (-2.0 * jnp.log(10000.0) / D_MODEL))
    pos_hi = (jax.lax.broadcasted_iota(jnp.int32, (hi_n, D_MODEL), 0) * 64
              + base).astype(jnp.float32)
    ang_a = pos_hi * div_hi
    sin_a, cos_a = jnp.sin(ang_a), jnp.cos(ang_a)

    j_lo = jax.lax.broadcasted_iota(jnp.int32, (64, D_MODEL), 1)
    k_lo = (j_lo // 2).astype(jnp.float32)
    div_lo = jnp.exp(k_lo * (-2.0 * jnp.log(10000.0) / D_MODEL))
    pos_lo = jax.lax.broadcasted_iota(jnp.int32, (64, D_MODEL), 0)
    phase = (j_lo % 2).astype(jnp.float32) * (0.5 * jnp.pi)
    ang_b = pos_lo.astype(jnp.float32) * div_lo + phase
    sin_b, cos_b = jnp.sin(ang_b), jnp.cos(ang_b)

    pe = (sin_a[:, None, :] * cos_b[None, :, :]
          + cos_a[:, None, :] * sin_b[None, :, :])
    o_ref[...] = pe.reshape(rows, D_MODEL)


def _make_pe(seq_len):
    rows = 512
    return pl.pallas_call(
        _pe_body,
        grid=(seq_len // rows,),
        out_specs=pl.BlockSpec((rows, D_MODEL), lambda i: (i, 0)),
        out_shape=jax.ShapeDtypeStruct((seq_len, D_MODEL), jnp.float32),
    )()


def _make_emb(B, V, D, seq_len):
    assert seq_len % _NW == 0
    s_w = seq_len // _NW
    Cs = 16
    assert s_w % Cs == 0
    nchunk = s_w // Cs
    R = B * Cs

    mesh = plsc.VectorSubcoreMesh(core_axis_name="c", subcore_axis_name="s")

    @functools.partial(
        pl.kernel,
        mesh=mesh,
        out_type=jax.ShapeDtypeStruct((B * seq_len, D), jnp.float32),
        scratch_types=[
            pltpu.VMEM((B, s_w), jnp.int32),
            pltpu.VMEM((R, D), jnp.float32),
            pltpu.VMEM((R, D), jnp.float32),
            pltpu.VMEM((Cs, D), jnp.float32),
            pltpu.VMEM((Cs, D), jnp.float32),
            pltpu.SemaphoreType.DMA,
            pltpu.SemaphoreType.DMA,
            pltpu.SemaphoreType.DMA,
            pltpu.SemaphoreType.DMA,
        ],
    )
    def emb(x_hbm, table_hbm, pe_hbm, out_hbm,
            idx_v, g0, g1, p0, p1, sg0, sg1, ss0, ss1):
        wid = lax.axis_index("s") * _NC + lax.axis_index("c")
        s_base = wid * s_w
        for b in range(B):
            pltpu.sync_copy(x_hbm.at[pl.ds(b * seq_len + s_base, s_w)],
                            idx_v.at[b])

        gbufs = (g0, g1)
        pbufs = (p0, p1)
        gsems = (sg0, sg1)
        ssems = (ss0, ss1)

        def issue(c, k):
            ds = []
            for b in range(B):
                ds.append(pltpu.async_copy(
                    table_hbm.at[idx_v.at[b, pl.ds(c * Cs, Cs)]],
                    gbufs[k].at[pl.ds(b * Cs, Cs)], gsems[k]))
            ds.append(pltpu.async_copy(
                pe_hbm.at[pl.ds(s_base + c * Cs, Cs)], pbufs[k], gsems[k]))
            return ds

        store_desc = [None, None]
        descs = [None, None]
        descs[0] = issue(0, 0)
        for c in range(nchunk):
            k = c & 1
            if c + 1 < nchunk:
                if store_desc[1 - k] is not None:
                    for sd in store_desc[1 - k]:
                        sd.wait()
                    store_desc[1 - k] = None
                descs[1 - k] = issue(c + 1, 1 - k)
            for d in descs[k]:
                d.wait()
            g, p = gbufs[k], pbufs[k]

            for b in range(B):
                @plsc.parallel_loop(0, Cs)
                def _row(r):
                    @plsc.parallel_loop(0, D, step=16, unroll=8)
                    def _col(j, b=b):
                        g[b * Cs + r, pl.ds(j, 16)] = (
                            g[b * Cs + r, pl.ds(j, 16)] + p[r, pl.ds(j, 16)])

            sds = []
            for b in range(B):
                sds.append(pltpu.async_copy(
                    g.at[pl.ds(b * Cs, Cs)],
                    out_hbm.at[pl.ds(b * seq_len + s_base + c * Cs, Cs)],
                    ssems[k]))
            store_desc[k] = sds
        for k in range(2):
            if store_desc[k] is not None:
                for sd in store_desc[k]:
                    sd.wait()

    return emb


def kernel(x, table):
    B, S = x.shape
    V, D = table.shape
    pe = _make_pe(S)
    xf = x.reshape(-1).astype(jnp.int32)
    out = _make_emb(B, V, D, S)(xf, table, pe)
    return out.reshape(B, S, D)


# speedup vs baseline: 2.4004x; 1.1809x over previous
import functools

import jax
import jax.numpy as jnp
from jax import lax
from jax.experimental import pallas as pl
from jax.experimental.pallas import tpu as pltpu
from jax.experimental.pallas import tpu_sc as plsc

D_MODEL = 768
MAX_S = 4096

try:
    _info = plsc.get_sparse_core_info()
    _NC, _NS = _info.num_cores, _info.num_subcores
except ValueError:
    _NC, _NS = 2, 16
_NW = _NC * _NS


def _pe_body(o_ref):
    rows = o_ref.shape[0]
    hi_n = rows // 64
    base = pl.program_id(0) * rows

    j_hi = jax.lax.broadcasted_iota(jnp.int32, (hi_n, D_MODEL), 1)
    k_hi = (j_hi // 2).astype(jnp.float32)
    div_hi = jnp.exp(k_hi * (-2.0 * jnp.log(10000.0) / D_MODEL))
    pos_hi = (jax.lax.broadcasted_iota(jnp.int32, (hi_n, D_MODEL), 0) * 64
              + base).astype(jnp.float32)
    ang_a = pos_hi * div_hi
    sin_a, cos_a = jnp.sin(ang_a), jnp.cos(ang_a)

    j_lo = jax.lax.broadcasted_iota(jnp.int32, (64, D_MODEL), 1)
    k_lo = (j_lo // 2).astype(jnp.float32)
    div_lo = jnp.exp(k_lo * (-2.0 * jnp.log(10000.0) / D_MODEL))
    pos_lo = jax.lax.broadcasted_iota(jnp.int32, (64, D_MODEL), 0)
    phase = (j_lo % 2).astype(jnp.float32) * (0.5 * jnp.pi)
    ang_b = pos_lo.astype(jnp.float32) * div_lo + phase
    sin_b, cos_b = jnp.sin(ang_b), jnp.cos(ang_b)

    pe = (sin_a[:, None, :] * cos_b[None, :, :]
          + cos_a[:, None, :] * sin_b[None, :, :])
    o_ref[...] = pe.reshape(rows, D_MODEL)


def _make_pe(seq_len):
    rows = 512
    return pl.pallas_call(
        _pe_body,
        grid=(seq_len // rows,),
        out_specs=pl.BlockSpec((rows, D_MODEL), lambda i: (i, 0)),
        out_shape=jax.ShapeDtypeStruct((seq_len, D_MODEL), jnp.float32),
    )()


def _make_emb(B, V, D, seq_len):
    assert seq_len % _NW == 0
    s_w = seq_len // _NW
    Cs = 8
    NBUF = 3
    assert s_w % Cs == 0
    nchunk = s_w // Cs
    R = B * Cs

    mesh = plsc.VectorSubcoreMesh(core_axis_name="c", subcore_axis_name="s")

    @functools.partial(
        pl.kernel,
        mesh=mesh,
        out_type=jax.ShapeDtypeStruct((B * seq_len, D), jnp.float32),
        scratch_types=(
            [pltpu.VMEM((B, s_w), jnp.int32)]
            + [pltpu.VMEM((R, D), jnp.float32) for _ in range(NBUF)]
            + [pltpu.VMEM((Cs, D), jnp.float32) for _ in range(NBUF)]
            + [pltpu.SemaphoreType.DMA for _ in range(2 * NBUF)]
        ),
    )
    def emb(x_hbm, table_hbm, pe_hbm, out_hbm, idx_v, *rest):
        gbufs = rest[:NBUF]
        pbufs = rest[NBUF:2 * NBUF]
        gsems = rest[2 * NBUF:3 * NBUF]
        ssems = rest[3 * NBUF:4 * NBUF]
        wid = lax.axis_index("s") * _NC + lax.axis_index("c")
        s_base = wid * s_w
        pltpu.sync_copy(x_hbm.at[:, pl.ds(s_base, s_w)], idx_v)

        def issue(c, k):
            ds = []
            for b in range(B):
                ds.append(pltpu.async_copy(
                    table_hbm.at[idx_v.at[b, pl.ds(c * Cs, Cs)]],
                    gbufs[k].at[pl.ds(b * Cs, Cs)], gsems[k]))
            ds.append(pltpu.async_copy(
                pe_hbm.at[pl.ds(s_base + c * Cs, Cs)], pbufs[k], gsems[k]))
            return ds

        store_desc = [None] * NBUF
        descs = [None] * NBUF
        for c in range(min(NBUF - 1, nchunk)):
            descs[c] = issue(c, c)
        for c in range(nchunk):
            k = c % NBUF
            cn = c + NBUF - 1
            if cn < nchunk:
                nk = cn % NBUF
                if store_desc[nk] is not None:
                    for sd in store_desc[nk]:
                        sd.wait()
                    store_desc[nk] = None
                descs[nk] = issue(cn, nk)
            for d in descs[k]:
                d.wait()
            g, p = gbufs[k], pbufs[k]

            @plsc.parallel_loop(0, Cs)
            def _row(r):
                @plsc.parallel_loop(0, D, step=16, unroll=4)
                def _col(j):
                    pv = p[r, pl.ds(j, 16)]
                    for b in range(B):
                        g[b * Cs + r, pl.ds(j, 16)] = (
                            g[b * Cs + r, pl.ds(j, 16)] + pv)

            sds = []
            for b in range(B):
                sds.append(pltpu.async_copy(
                    g.at[pl.ds(b * Cs, Cs)],
                    out_hbm.at[pl.ds(b * seq_len + s_base + c * Cs, Cs)],
                    ssems[k]))
            store_desc[k] = sds
        for k in range(NBUF):
            if store_desc[k] is not None:
                for sd in store_desc[k]:
                    sd.wait()

    return emb


def kernel(x, table):
    B, S = x.shape
    V, D = table.shape
    pe = _make_pe(S)
    out = _make_emb(B, V, D, S)(x.astype(jnp.int32), table, pe)
    return out.reshape(B, S, D)
